# Initial kernel scaffold; baseline (speedup 1.0000x reference)
import jax
import jax.numpy as jnp
from jax.experimental import pallas as pl


def kernel(x, edge_index, W, b, w1, w2, w3):
    raise NotImplementedError("write your pallas kernel here")


# speedup vs baseline: 1.2704x; 1.2704x over previous
import functools

import jax
import jax.numpy as jnp
from jax import lax
from jax.experimental import pallas as pl
from jax.experimental.pallas import tpu as pltpu, tpu_sc as plsc

N = 16
E = 24
D = 512
L = 16
NC = 2
NS = 16

_f32 = jnp.float32
_i32 = jnp.int32


def _splat(v, lane):
    idx = jnp.full((L,), lane, dtype=_i32)
    return jnp.take_along_axis(v, idx, axis=0)


def _vgather(v, idx):
    return jnp.take_along_axis(v, idx, axis=0)


def _gat_body(x_hbm, ei_hbm, w_hbm, par_hbm, out_hbm,
              xs, wv, eiv, pvv, ypart, yall, shared, agg):
    c = lax.axis_index("c")
    s = lax.axis_index("s")

    ybase = s * 32

    pltpu.sync_copy(x_hbm.at[:, pl.ds(ybase, 32)], xs)
    pltpu.sync_copy(w_hbm.at[0, pl.ds(ybase, 32)], wv)
    pltpu.sync_copy(ei_hbm, eiv)
    pltpu.sync_copy(par_hbm, pvv)

    iota = lax.iota(_i32, L)
    fiota = iota.astype(_f32)
    zero = fiota * 0.0

    w0 = wv[0:16]
    w1v = wv[16:32]
    yacc = zero
    for j in range(32):
        col = plsc.load_gather(xs, [iota, jnp.full((L,), j, dtype=_i32)])
        wj = _splat(w0 if j < 16 else w1v, j % 16)
        yacc = yacc + col * wj
    ypart[...] = yacc

    pltpu.sync_copy(ypart, shared.at[s])
    plsc.subcore_barrier()
    pltpu.sync_copy(shared, yall)
    y = zero
    for i in range(NS):
        y = y + yall[i, :]

    pv = pvv[...]
    b_spl = _splat(pv, 0)
    w1_spl = _splat(pv, 1)
    w2_spl = _splat(pv, 2)
    w3_spl = _splat(pv, 3)

    src0 = eiv[0, 0:16]
    src1 = eiv[0, 16:32]
    dst0 = eiv[1, 0:16]
    dst1 = eiv[1, 16:32]

    learned = b_spl
    for e in range(E):
        srcv = src0 if e < 16 else src1
        dstv = dst0 if e < 16 else dst1
        lane = e % 16
        s_spl = _splat(srcv, lane)
        d_spl = _splat(dstv, lane)
        msg = _vgather(y, s_spl) - _vgather(y, d_spl) + b_spl
        learned = jnp.where(iota == d_spl, learned + msg, learned)

    fes2 = jnp.where((iota % 2) == 0, w2_spl, w3_spl) + learned

    cbase = c * 16
    for r in range(N):
        agg[r, :] = zero
    for e in range(E):
        srcv = src0 if e < 16 else src1
        dstv = dst0 if e < 16 else dst1
        lane = e % 16
        s_spl = _splat(srcv, lane)
        d_spl = _splat(dstv, lane)
        xrow = plsc.load_gather(xs, [s_spl, iota + cbase])
        scale = w1_spl if e < 8 else _splat(fes2, e - 8)
        plsc.addupdate_scatter(agg, [d_spl, iota], xrow * scale)

    for r in range(N):
        agg[r, :] = jnp.maximum(agg[r, :], 0.0)
    pltpu.sync_copy(agg, out_hbm.at[:, pl.ds(ybase + cbase, 16)])


_gat_kernel = functools.partial(
    pl.kernel,
    out_type=jax.ShapeDtypeStruct((N, D), _f32),
    mesh=plsc.VectorSubcoreMesh(core_axis_name="c", subcore_axis_name="s"),
    compiler_params=pltpu.CompilerParams(
        use_tc_tiling_on_sc=False, needs_layout_passes=False
    ),
    scratch_types=[
        pltpu.VMEM((N, 32), _f32),
        pltpu.VMEM((32,), _f32),
        pltpu.VMEM((2, 32), _i32),
        pltpu.VMEM((L,), _f32),
        pltpu.VMEM((L,), _f32),
        pltpu.VMEM((NS, L), _f32),
        pltpu.VMEM_SHARED((NS, L), _f32),
        pltpu.VMEM((N, L), _f32),
    ],
)(_gat_body)


@jax.jit
def kernel(x, edge_index, W, b, w1, w2, w3):
    ei_pad = jnp.zeros((2, 32), dtype=_i32).at[:, :E].set(edge_index)
    params = jnp.zeros((L,), dtype=_f32)
    params = params.at[0].set(b[0]).at[1].set(w1[0]).at[2].set(w2[0]).at[3].set(w3[0])
    return _gat_kernel(x, ei_pad, W, params)


# speedup vs baseline: 1.5365x; 1.2095x over previous
import functools

import jax
import jax.numpy as jnp
from jax import lax
from jax.experimental import pallas as pl
from jax.experimental.pallas import tpu as pltpu, tpu_sc as plsc

N = 16
E = 24
D = 512
L = 16
NS = 16

_f32 = jnp.float32
_i32 = jnp.int32


def _splat(v, lane):
    idx = jnp.full((L,), lane, dtype=_i32)
    return jnp.take_along_axis(v, idx, axis=0)


def _gat_body(x_hbm, ei_hbm, w_hbm, b_hbm, w1_hbm, w2_hbm, w3_hbm, out_hbm,
              xs, wv, eiv, pbuf, ypart, yall, shared, agg, sem):
    c = lax.axis_index("c")
    s = lax.axis_index("s")
    ybase = s * 32
    cbase = c * 16

    cx = pltpu.async_copy(x_hbm.at[:, pl.ds(ybase, 32)], xs, sem)
    cw = pltpu.async_copy(w_hbm.at[0, pl.ds(ybase, 32)], wv, sem)
    ce = pltpu.async_copy(ei_hbm, eiv, sem)
    cp0 = pltpu.async_copy(b_hbm, pbuf.at[0, pl.ds(0, 1)], sem)
    cp1 = pltpu.async_copy(w1_hbm, pbuf.at[1, pl.ds(0, 1)], sem)
    cp2 = pltpu.async_copy(w2_hbm, pbuf.at[2, pl.ds(0, 1)], sem)
    cp3 = pltpu.async_copy(w3_hbm, pbuf.at[3, pl.ds(0, 1)], sem)

    iota = lax.iota(_i32, L)
    zero = iota.astype(_f32) * 0.0

    cx.wait()
    cw.wait()
    w0 = wv[0:16]
    w1v = wv[16:32]
    yacc = zero
    for j in range(32):
        col = plsc.load_gather(xs, [iota, jnp.full((L,), j, dtype=_i32)])
        wj = _splat(w0 if j < 16 else w1v, j % 16)
        yacc = yacc + col * wj
    ypart[...] = yacc

    pltpu.sync_copy(ypart, shared.at[s])
    plsc.subcore_barrier()
    pltpu.sync_copy(shared, yall)
    y = zero
    for i in range(NS):
        y = y + yall[i, :]

    ce.wait()
    cp0.wait()
    cp1.wait()
    cp2.wait()
    cp3.wait()
    b_spl = _splat(pbuf[0, 0:16], 0)
    w1_spl = _splat(pbuf[1, 0:16], 0)
    w2_spl = _splat(pbuf[2, 0:16], 0)
    w3_spl = _splat(pbuf[3, 0:16], 0)

    src_a = eiv[0, 0:16]
    src_b = eiv[0, 8:24]
    dst_a = eiv[1, 0:16]
    dst_b = eiv[1, 8:24]

    learned = b_spl
    for e in range(E):
        srcv, dstv, lane = (src_a, dst_a, e) if e < 16 else (src_b, dst_b, e - 8)
        s_spl = _splat(srcv, lane)
        d_spl = _splat(dstv, lane)
        msg = jnp.take_along_axis(y, s_spl, axis=0) \
            - jnp.take_along_axis(y, d_spl, axis=0) + b_spl
        learned = jnp.where(iota == d_spl, learned + msg, learned)

    fes2 = jnp.where((iota % 2) == 0, w2_spl, w3_spl) + learned

    for r in range(N):
        agg[r, :] = zero
    for e in range(E):
        srcv, dstv, lane = (src_a, dst_a, e) if e < 16 else (src_b, dst_b, e - 8)
        s_spl = _splat(srcv, lane)
        d_spl = _splat(dstv, lane)
        xrow = plsc.load_gather(xs, [s_spl, iota + cbase])
        scale = w1_spl if e < 8 else _splat(fes2, e - 8)
        plsc.addupdate_scatter(agg, [d_spl, iota], xrow * scale)

    for r in range(N):
        agg[r, :] = jnp.maximum(agg[r, :], 0.0)
    pltpu.sync_copy(agg, out_hbm.at[:, pl.ds(ybase + cbase, 16)])


_gat_kernel = functools.partial(
    pl.kernel,
    out_type=jax.ShapeDtypeStruct((N, D), _f32),
    mesh=plsc.VectorSubcoreMesh(core_axis_name="c", subcore_axis_name="s"),
    compiler_params=pltpu.CompilerParams(
        use_tc_tiling_on_sc=False, needs_layout_passes=False
    ),
    scratch_types=[
        pltpu.VMEM((N, 32), _f32),
        pltpu.VMEM((32,), _f32),
        pltpu.VMEM((2, E), _i32),
        pltpu.VMEM((4, L), _f32),
        pltpu.VMEM((L,), _f32),
        pltpu.VMEM((NS, L), _f32),
        pltpu.VMEM_SHARED((NS, L), _f32),
        pltpu.VMEM((N, L), _f32),
        pltpu.SemaphoreType.DMA,
    ],
)(_gat_body)


@jax.jit
def kernel(x, edge_index, W, b, w1, w2, w3):
    return _gat_kernel(x, edge_index, W, b, w1, w2, w3)


# speedup vs baseline: 1.5700x; 1.0218x over previous
import functools

import jax
import jax.numpy as jnp
from jax import lax
from jax.experimental import pallas as pl
from jax.experimental.pallas import tpu as pltpu, tpu_sc as plsc

N = 16
E = 24
D = 512
L = 16
NS = 16

_f32 = jnp.float32
_i32 = jnp.int32


def _splat(v, lane):
    idx = jnp.full((L,), lane, dtype=_i32)
    return jnp.take_along_axis(v, idx, axis=0)


def _gat_body(x_hbm, ei_hbm, w_hbm, b_hbm, w1_hbm, w2_hbm, w3_hbm, out_hbm,
              xs, wv, eiv, pbuf, ypart, yall, learned_ref, shared, agg, sem):
    c = lax.axis_index("c")
    s = lax.axis_index("s")
    ybase = s * 32
    cbase = c * 16

    cx = pltpu.async_copy(x_hbm.at[:, pl.ds(ybase, 32)], xs, sem)
    cw = pltpu.async_copy(w_hbm.at[0, pl.ds(ybase, 32)], wv, sem)
    ce = pltpu.async_copy(ei_hbm, eiv, sem)
    cp0 = pltpu.async_copy(b_hbm, pbuf.at[0, pl.ds(0, 1)], sem)
    cp1 = pltpu.async_copy(w1_hbm, pbuf.at[1, pl.ds(0, 1)], sem)
    cp2 = pltpu.async_copy(w2_hbm, pbuf.at[2, pl.ds(0, 1)], sem)
    cp3 = pltpu.async_copy(w3_hbm, pbuf.at[3, pl.ds(0, 1)], sem)

    iota = lax.iota(_i32, L)
    zero = iota.astype(_f32) * 0.0

    for r in range(N):
        agg[r, :] = zero

    cx.wait()
    cw.wait()
    w0 = wv[0:16]
    w1v = wv[16:32]
    yacc = zero
    for j in range(32):
        col = plsc.load_gather(xs, [iota, jnp.full((L,), j, dtype=_i32)])
        wj = _splat(w0 if j < 16 else w1v, j % 16)
        yacc = yacc + col * wj
    ypart[...] = yacc
    pltpu.sync_copy(ypart, shared.at[s])

    ce.wait()
    cp0.wait()
    cp1.wait()
    cp2.wait()
    cp3.wait()
    b_spl = _splat(pbuf[0, 0:16], 0)
    w1_spl = _splat(pbuf[1, 0:16], 0)
    w2_spl = _splat(pbuf[2, 0:16], 0)
    w3_spl = _splat(pbuf[3, 0:16], 0)

    src_a = eiv[0, 0:16]
    src_b = eiv[0, 8:24]
    dst_a = eiv[1, 0:16]
    dst_b = eiv[1, 8:24]

    for e in range(8):
        s_spl = _splat(src_a, e)
        d_spl = _splat(dst_a, e)
        xrow = plsc.load_gather(xs, [s_spl, iota + cbase])
        plsc.addupdate_scatter(agg, [d_spl, iota], xrow * w1_spl)

    plsc.subcore_barrier()
    pltpu.sync_copy(shared, yall)
    y = zero
    for i in range(NS):
        y = y + yall[i, :]

    msg_a = jnp.take_along_axis(y, src_a, axis=0) \
        - jnp.take_along_axis(y, dst_a, axis=0) + b_spl
    msg_b = jnp.take_along_axis(y, src_b, axis=0) \
        - jnp.take_along_axis(y, dst_b, axis=0) + b_spl
    learned_ref[...] = b_spl
    plsc.addupdate_scatter(learned_ref, [dst_a], msg_a)
    plsc.addupdate_scatter(learned_ref, [dst_b], msg_b, mask=iota >= 8)
    learned = learned_ref[...]

    fes2 = jnp.where((iota % 2) == 0, w2_spl, w3_spl) + learned

    for e in range(8, E):
        srcv, dstv, lane = (src_a, dst_a, e) if e < 16 else (src_b, dst_b, e - 8)
        s_spl = _splat(srcv, lane)
        d_spl = _splat(dstv, lane)
        xrow = plsc.load_gather(xs, [s_spl, iota + cbase])
        scale = _splat(fes2, e - 8)
        plsc.addupdate_scatter(agg, [d_spl, iota], xrow * scale)

    for r in range(N):
        agg[r, :] = jnp.maximum(agg[r, :], 0.0)
    pltpu.sync_copy(agg, out_hbm.at[:, pl.ds(ybase + cbase, 16)])


_gat_kernel = functools.partial(
    pl.kernel,
    out_type=jax.ShapeDtypeStruct((N, D), _f32),
    mesh=plsc.VectorSubcoreMesh(core_axis_name="c", subcore_axis_name="s"),
    compiler_params=pltpu.CompilerParams(
        use_tc_tiling_on_sc=False,
        needs_layout_passes=False,
        disable_bounds_checks=True,
        disable_semaphore_checks=True,
    ),
    scratch_types=[
        pltpu.VMEM((N, 32), _f32),
        pltpu.VMEM((32,), _f32),
        pltpu.VMEM((2, E), _i32),
        pltpu.VMEM((4, L), _f32),
        pltpu.VMEM((L,), _f32),
        pltpu.VMEM((NS, L), _f32),
        pltpu.VMEM((L,), _f32),
        pltpu.VMEM_SHARED((NS, L), _f32),
        pltpu.VMEM((N, L), _f32),
        pltpu.SemaphoreType.DMA,
    ],
)(_gat_body)


@jax.jit
def kernel(x, edge_index, W, b, w1, w2, w3):
    return _gat_kernel(x, edge_index, W, b, w1, w2, w3)


# speedup vs baseline: 1.6617x; 1.0584x over previous
import functools

import jax
import jax.numpy as jnp
from jax import lax
from jax.experimental import pallas as pl
from jax.experimental.pallas import tpu as pltpu, tpu_sc as plsc

N = 16
E = 24
D = 512
L = 16
NS = 16

_f32 = jnp.float32
_i32 = jnp.int32


def _splat(v, lane):
    idx = jnp.full((L,), lane, dtype=_i32)
    return jnp.take_along_axis(v, idx, axis=0)


def _gat_body(x_hbm, src_hbm, dst_hbm, w_hbm, b_hbm, w1_hbm, w2_hbm, w3_hbm,
              out_hbm,
              xs, wv, srcv, dstv, pbuf, ypart, yall, learned_ref, shared, agg,
              sem):
    c = lax.axis_index("c")
    s = lax.axis_index("s")
    ybase = s * 32
    ct = s // 4
    coff = (s % 4) * 32
    obase = c * 16

    cx = pltpu.async_copy(x_hbm.at[:, ct, :, pl.ds(coff, 32)], xs, sem)
    cw = pltpu.async_copy(w_hbm.at[0, pl.ds(ybase, 32)], wv, sem)
    ces = pltpu.async_copy(src_hbm, srcv, sem)
    ced = pltpu.async_copy(dst_hbm, dstv, sem)
    cp0 = pltpu.async_copy(b_hbm, pbuf.at[0, pl.ds(0, 1)], sem)
    cp1 = pltpu.async_copy(w1_hbm, pbuf.at[1, pl.ds(0, 1)], sem)
    cp2 = pltpu.async_copy(w2_hbm, pbuf.at[2, pl.ds(0, 1)], sem)
    cp3 = pltpu.async_copy(w3_hbm, pbuf.at[3, pl.ds(0, 1)], sem)

    iota = lax.iota(_i32, L)
    idiv8 = iota >> 3
    imod8 = iota & 7
    zero = iota.astype(_f32) * 0.0

    for r0 in range(2):
        for r1 in range(8):
            agg[r0, r1, :] = zero

    cx.wait()
    cw.wait()
    w0 = wv[0:16]
    w1v = wv[16:32]
    yacc = zero
    for j in range(32):
        col = plsc.load_gather(xs, [idiv8, imod8, jnp.full((L,), j, dtype=_i32)])
        wj = _splat(w0 if j < 16 else w1v, j % 16)
        yacc = yacc + col * wj
    ypart[...] = yacc
    pltpu.sync_copy(ypart, shared.at[s])

    ces.wait()
    ced.wait()
    cp0.wait()
    cp1.wait()
    cp2.wait()
    cp3.wait()
    b_spl = _splat(pbuf[0, 0:16], 0)
    w1_spl = _splat(pbuf[1, 0:16], 0)
    w2_spl = _splat(pbuf[2, 0:16], 0)
    w3_spl = _splat(pbuf[3, 0:16], 0)

    src_a = srcv[0:16]
    src_b = srcv[8:24]
    dst_a = dstv[0:16]
    dst_b = dstv[8:24]

    for e in range(8):
        s_spl = _splat(src_a, e)
        d_spl = _splat(dst_a, e)
        xrow = plsc.load_gather(xs, [s_spl >> 3, s_spl & 7, iota + obase])
        plsc.addupdate_scatter(agg, [d_spl >> 3, d_spl & 7, iota],
                               xrow * w1_spl)

    plsc.subcore_barrier()
    pltpu.sync_copy(shared, yall)
    y = zero
    for i in range(NS):
        y = y + yall[i, :]

    msg_a = jnp.take_along_axis(y, src_a, axis=0) \
        - jnp.take_along_axis(y, dst_a, axis=0) + b_spl
    msg_b = jnp.take_along_axis(y, src_b, axis=0) \
        - jnp.take_along_axis(y, dst_b, axis=0) + b_spl
    learned_ref[...] = b_spl
    plsc.addupdate_scatter(learned_ref, [dst_a], msg_a)
    plsc.addupdate_scatter(learned_ref, [dst_b], msg_b, mask=iota >= 8)
    learned = learned_ref[...]

    fes2 = jnp.where((iota % 2) == 0, w2_spl, w3_spl) + learned

    for e in range(8, E):
        srca, dsta, lane = (src_a, dst_a, e) if e < 16 else (src_b, dst_b, e - 8)
        s_spl = _splat(srca, lane)
        d_spl = _splat(dsta, lane)
        xrow = plsc.load_gather(xs, [s_spl >> 3, s_spl & 7, iota + obase])
        scale = _splat(fes2, e - 8)
        plsc.addupdate_scatter(agg, [d_spl >> 3, d_spl & 7, iota],
                               xrow * scale)

    for r0 in range(2):
        for r1 in range(8):
            agg[r0, r1, :] = jnp.maximum(agg[r0, r1, :], 0.0)
    pltpu.sync_copy(agg, out_hbm.at[:, ct, :, pl.ds(coff + obase, 16)])


_gat_kernel = functools.partial(
    pl.kernel,
    out_type=jax.ShapeDtypeStruct((2, 4, 8, 128), _f32),
    mesh=plsc.VectorSubcoreMesh(core_axis_name="c", subcore_axis_name="s"),
    compiler_params=pltpu.CompilerParams(
        use_tc_tiling_on_sc=False,
        needs_layout_passes=False,
        disable_bounds_checks=True,
        disable_semaphore_checks=True,
    ),
    scratch_types=[
        pltpu.VMEM((2, 8, 32), _f32),
        pltpu.VMEM((32,), _f32),
        pltpu.VMEM((E,), _i32),
        pltpu.VMEM((E,), _i32),
        pltpu.VMEM((4, L), _f32),
        pltpu.VMEM((L,), _f32),
        pltpu.VMEM((NS, L), _f32),
        pltpu.VMEM((L,), _f32),
        pltpu.VMEM_SHARED((NS, L), _f32),
        pltpu.VMEM((2, 8, L), _f32),
        pltpu.SemaphoreType.DMA,
    ],
)(_gat_body)


@jax.jit
def kernel(x, edge_index, W, b, w1, w2, w3):
    x4 = x.reshape(2, 8, 4, 128).transpose(0, 2, 1, 3)
    out4 = _gat_kernel(x4, edge_index[0], edge_index[1], W, b, w1, w2, w3)
    return out4.transpose(0, 2, 1, 3).reshape(N, D)


# speedup vs baseline: 1.7524x; 1.0546x over previous
import functools

import jax
import jax.numpy as jnp
from jax import lax
from jax.experimental import pallas as pl
from jax.experimental.pallas import tpu as pltpu, tpu_sc as plsc

N = 16
E = 24
D = 512
L = 16
NS = 16

_f32 = jnp.float32
_i32 = jnp.int32


def _splat(v, lane):
    idx = jnp.full((L,), lane, dtype=_i32)
    return jnp.take_along_axis(v, idx, axis=0)


def _gat_body(x_hbm, src_hbm, dst_hbm, w_hbm, b_hbm, w1_hbm, w2_hbm, w3_hbm,
              out_hbm,
              xs, wv, srcv, dstv, pbuf, ypart, yall, learned_ref, shared, agg,
              sem):
    s = lax.axis_index("s")
    ybase = s * 32
    ct = s // 4
    coff = (s % 4) * 32

    cx = pltpu.async_copy(x_hbm.at[:, ct, :, pl.ds(coff, 32)], xs, sem)
    cw = pltpu.async_copy(w_hbm.at[0, pl.ds(ybase, 32)], wv, sem)
    ces = pltpu.async_copy(src_hbm, srcv, sem)
    ced = pltpu.async_copy(dst_hbm, dstv, sem)
    cp0 = pltpu.async_copy(b_hbm, pbuf.at[0, pl.ds(0, 1)], sem)
    cp1 = pltpu.async_copy(w1_hbm, pbuf.at[1, pl.ds(0, 1)], sem)
    cp2 = pltpu.async_copy(w2_hbm, pbuf.at[2, pl.ds(0, 1)], sem)
    cp3 = pltpu.async_copy(w3_hbm, pbuf.at[3, pl.ds(0, 1)], sem)

    iota = lax.iota(_i32, L)
    idiv8 = iota >> 3
    imod8 = iota & 7
    zero = iota.astype(_f32) * 0.0

    for r0 in range(2):
        for r1 in range(8):
            agg[r0, r1, 0:16] = zero
            agg[r0, r1, 16:32] = zero

    cx.wait()
    cw.wait()
    w0 = wv[0:16]
    w1v = wv[16:32]
    yacc = zero
    for j in range(32):
        col = plsc.load_gather(xs, [idiv8, imod8, jnp.full((L,), j, dtype=_i32)])
        wj = _splat(w0 if j < 16 else w1v, j % 16)
        yacc = yacc + col * wj
    ypart[...] = yacc
    pltpu.sync_copy(ypart, shared.at[s])

    ces.wait()
    ced.wait()
    cp0.wait()
    cp1.wait()
    cp2.wait()
    cp3.wait()
    b_spl = _splat(pbuf[0, 0:16], 0)
    w1_spl = _splat(pbuf[1, 0:16], 0)
    w2_spl = _splat(pbuf[2, 0:16], 0)
    w3_spl = _splat(pbuf[3, 0:16], 0)

    src_a = srcv[0:16]
    src_b = srcv[8:24]
    dst_a = dstv[0:16]
    dst_b = dstv[8:24]

    for e in range(8):
        s_spl = _splat(src_a, e)
        d_spl = _splat(dst_a, e)
        sd, sm = s_spl >> 3, s_spl & 7
        dd, dm = d_spl >> 3, d_spl & 7
        for o in (0, 16):
            xrow = plsc.load_gather(xs, [sd, sm, iota + o])
            plsc.addupdate_scatter(agg, [dd, dm, iota + o], xrow * w1_spl)

    plsc.subcore_barrier()
    pltpu.sync_copy(shared, yall)
    y = zero
    for i in range(NS):
        y = y + yall[i, :]

    msg_a = jnp.take_along_axis(y, src_a, axis=0) \
        - jnp.take_along_axis(y, dst_a, axis=0) + b_spl
    msg_b = jnp.take_along_axis(y, src_b, axis=0) \
        - jnp.take_along_axis(y, dst_b, axis=0) + b_spl
    learned_ref[...] = b_spl
    plsc.addupdate_scatter(learned_ref, [dst_a], msg_a)
    plsc.addupdate_scatter(learned_ref, [dst_b], msg_b, mask=iota >= 8)
    learned = learned_ref[...]

    fes2 = jnp.where((iota % 2) == 0, w2_spl, w3_spl) + learned

    for e in range(8, E):
        srca, dsta, lane = (src_a, dst_a, e) if e < 16 else (src_b, dst_b, e - 8)
        s_spl = _splat(srca, lane)
        d_spl = _splat(dsta, lane)
        sd, sm = s_spl >> 3, s_spl & 7
        dd, dm = d_spl >> 3, d_spl & 7
        scale = _splat(fes2, e - 8)
        for o in (0, 16):
            xrow = plsc.load_gather(xs, [sd, sm, iota + o])
            plsc.addupdate_scatter(agg, [dd, dm, iota + o], xrow * scale)

    for r0 in range(2):
        for r1 in range(8):
            agg[r0, r1, 0:16] = jnp.maximum(agg[r0, r1, 0:16], 0.0)
            agg[r0, r1, 16:32] = jnp.maximum(agg[r0, r1, 16:32], 0.0)
    pltpu.sync_copy(agg, out_hbm.at[:, ct, :, pl.ds(coff, 32)])


_gat_kernel = functools.partial(
    pl.kernel,
    out_type=jax.ShapeDtypeStruct((2, 4, 8, 128), _f32),
    mesh=plsc.VectorSubcoreMesh(core_axis_name="c", subcore_axis_name="s",
                                num_cores=1),
    compiler_params=pltpu.CompilerParams(
        use_tc_tiling_on_sc=False,
        needs_layout_passes=False,
        disable_bounds_checks=True,
        disable_semaphore_checks=True,
    ),
    scratch_types=[
        pltpu.VMEM((2, 8, 32), _f32),
        pltpu.VMEM((32,), _f32),
        pltpu.VMEM((E,), _i32),
        pltpu.VMEM((E,), _i32),
        pltpu.VMEM((4, L), _f32),
        pltpu.VMEM((L,), _f32),
        pltpu.VMEM((NS, L), _f32),
        pltpu.VMEM((L,), _f32),
        pltpu.VMEM_SHARED((NS, L), _f32),
        pltpu.VMEM((2, 8, 32), _f32),
        pltpu.SemaphoreType.DMA,
    ],
)(_gat_body)


@jax.jit
def kernel(x, edge_index, W, b, w1, w2, w3):
    x4 = x.reshape(2, 8, 4, 128).transpose(0, 2, 1, 3)
    out4 = _gat_kernel(x4, edge_index[0], edge_index[1], W, b, w1, w2, w3)
    return out4.transpose(0, 2, 1, 3).reshape(N, D)
